# baseline (speedup 1.0000x reference)
import jax
import jax.numpy as jnp
from jax import lax
from jax.experimental import pallas as pl
from jax.experimental.pallas import tpu as pltpu
from jax.experimental.pallas import tpu_sc as plsc

_N = 10000
_D_IN = 128
_HID = 256
_OUT = 64
_E = 320000

_NT = 16
_NP = 10240
_RT = _NP // _NT
_B = 125
_ET = _E // _NT
_NCH = _ET // _B
_IB = 40
_NB = _NCH // _IB

_BN = 1000


def _dot(a, b):
    return jnp.dot(a, b, preferred_element_type=jnp.float32)


def _proj_body(x_ref, w_ref, b_ref, h0_ref, h1_ref):
    h = _dot(x_ref[0], w_ref[0]) + b_ref[0]
    h = jnp.maximum(h, 0.0)
    h0_ref[0] = h[:, :128]
    h1_ref[0] = h[:, 128:]


def _proj(x2, w2, b2):
    return pl.pallas_call(
        _proj_body,
        grid=(2, _N // _BN),
        in_specs=[
            pl.BlockSpec((1, _BN, _D_IN), lambda t, i: (t, i, 0)),
            pl.BlockSpec((1, _D_IN, _HID), lambda t, i: (t, 0, 0)),
            pl.BlockSpec((1, 1, _HID), lambda t, i: (t, 0, 0)),
        ],
        out_specs=[
            pl.BlockSpec((1, _BN, 128), lambda t, i: (t, i, 0)),
            pl.BlockSpec((1, _BN, 128), lambda t, i: (t, i, 0)),
        ],
        out_shape=[jax.ShapeDtypeStruct((2, _N, 128), jnp.float32)] * 2,
    )(x2, w2, b2)


def _gather_scatter_dir(s, table, src3, dst3, sidx, didx, rows, acc, sems):
    gsem = (sems[0], sems[1])
    ssem = (sems[2], sems[3])

    def g_fire(ch, b):
        pltpu.async_copy(table.at[sidx.at[ch]], rows.at[b], gsem[b])

    def g_wait(ch, b):
        pltpu.make_async_copy(table.at[sidx.at[ch]], rows.at[b],
                              gsem[b]).wait()

    def s_fire(ch, b):
        pltpu.async_copy(rows.at[b], acc.at[didx.at[ch]], ssem[b], add=True)

    def s_wait(ch, b):
        pltpu.make_async_copy(rows.at[b], acc.at[didx.at[ch]],
                              ssem[b]).wait()

    def blk(k, carry):
        pltpu.sync_copy(src3.at[s * _NB + k], sidx)
        pltpu.sync_copy(dst3.at[s * _NB + k], didx)
        g_fire(0, 0)
        g_fire(1, 1)
        g_wait(0, 0)
        s_fire(0, 0)
        s_wait(0, 0)
        g_fire(2, 0)
        g_wait(1, 1)
        s_fire(1, 1)

        def pair(i, c2):
            for b in (0, 1):
                ch = i * 2 + b
                nb = 1 - b

                @pl.when(ch + 1 < _IB)
                def _():
                    s_wait(ch - 1, nb)
                    g_fire(ch + 1, nb)

                g_wait(ch, b)
                s_fire(ch, b)
            return c2

        lax.fori_loop(1, _IB // 2, pair, 0)
        s_wait(_IB - 2, 0)
        s_wait(_IB - 1, 1)
        return carry

    lax.fori_loop(0, _NB, blk, 0)


_CB = 40


def _count_dir(s, dst3, didx, ones_v, acc, sem):
    def blk(k, carry):
        pltpu.sync_copy(dst3.at[s * _NB + k], didx)

        def burst(i, c2):
            for j in range(_CB):
                pltpu.async_copy(ones_v, acc.at[didx.at[i * _CB + j]], sem,
                                 add=True)
            for j in range(_CB):
                pltpu.make_async_copy(ones_v, acc.at[didx.at[i * _CB + j]],
                                      sem).wait()
            return c2

        lax.fori_loop(0, _IB // _CB, burst, 0)
        return carry

    lax.fori_loop(0, _NB, blk, 0)


def _zero_rows(zfeat, acc, r0):
    pltpu.sync_copy(zfeat.at[pl.ds(r0, _RT), :], acc.at[pl.ds(r0, _RT), :])


def _writeout(acc, out, r0):
    pltpu.sync_copy(acc.at[pl.ds(r0, _RT), :], out.at[pl.ds(r0, _RT), :])


def _agg1_body(h0, h1, s_uwN, d_uw, s_wu, d_wu, zfeat,
               so0, so1,
               acc, rows, sidx, didx, sem0, sem1, sem2, sem3):
    c = lax.axis_index("c")
    s = lax.axis_index("s")
    r0 = s * _RT
    sems = (sem0, sem1, sem2, sem3)

    _zero_rows(zfeat, acc, r0)
    plsc.subcore_barrier()

    @pl.when(c == 0)
    def _():
        _gather_scatter_dir(s, h0, s_uwN, d_uw, sidx, didx, rows, acc, sems)

    @pl.when(c == 1)
    def _():
        _gather_scatter_dir(s, h1, s_uwN, d_uw, sidx, didx, rows, acc, sems)

    plsc.subcore_barrier()

    @pl.when(c == 0)
    def _():
        _writeout(acc, so0.at[0], r0)

    @pl.when(c == 1)
    def _():
        _writeout(acc, so1.at[0], r0)

    _zero_rows(zfeat, acc, r0)
    plsc.subcore_barrier()

    @pl.when(c == 0)
    def _():
        _gather_scatter_dir(s, h0, s_wu, d_wu, sidx, didx, rows, acc, sems)

    @pl.when(c == 1)
    def _():
        _gather_scatter_dir(s, h1, s_wu, d_wu, sidx, didx, rows, acc, sems)

    plsc.subcore_barrier()

    @pl.when(c == 0)
    def _():
        _writeout(acc, so0.at[1], r0)

    @pl.when(c == 1)
    def _():
        _writeout(acc, so1.at[1], r0)


def _make_agg1():
    mesh = plsc.VectorSubcoreMesh(core_axis_name="c", subcore_axis_name="s")
    out_type = [jax.ShapeDtypeStruct((2, _NP, 128), jnp.float32)] * 2
    scratch = [
        pltpu.VMEM_SHARED((_NP, 128), jnp.float32),
        pltpu.VMEM((2, _B, 128), jnp.float32),
        pltpu.VMEM((_IB, _B), jnp.int32),
        pltpu.VMEM((_IB, _B), jnp.int32),
        pltpu.SemaphoreType.DMA,
        pltpu.SemaphoreType.DMA,
        pltpu.SemaphoreType.DMA,
        pltpu.SemaphoreType.DMA,
    ]
    return pl.kernel(
        _agg1_body,
        out_type=out_type,
        mesh=mesh,
        scratch_types=scratch,
    )


def _cnt_body(d_uw, d_wu, zfeat, ones_h,
              cnt,
              acc, didx, ones_v, sem0):
    c = lax.axis_index("c")
    s = lax.axis_index("s")
    r0 = s * _RT

    pltpu.sync_copy(ones_h, ones_v)
    _zero_rows(zfeat, acc, r0)
    plsc.subcore_barrier()

    @pl.when(c == 0)
    def _():
        _count_dir(s, d_uw, didx, ones_v, acc, sem0)

    @pl.when(c == 1)
    def _():
        _count_dir(s, d_wu, didx, ones_v, acc, sem0)

    plsc.subcore_barrier()

    @pl.when(c == 0)
    def _():
        _writeout(acc, cnt.at[0], r0)

    @pl.when(c == 1)
    def _():
        _writeout(acc, cnt.at[1], r0)


def _make_cnt():
    mesh = plsc.VectorSubcoreMesh(core_axis_name="c", subcore_axis_name="s")
    out_type = jax.ShapeDtypeStruct((2, _NP, 128), jnp.float32)
    scratch = [
        pltpu.VMEM_SHARED((_NP, 128), jnp.float32),
        pltpu.VMEM((_IB, _B), jnp.int32),
        pltpu.VMEM((_B, 128), jnp.float32),
        pltpu.SemaphoreType.DMA,
    ]
    return pl.kernel(
        _cnt_body,
        out_type=out_type,
        mesh=mesh,
        scratch_types=scratch,
    )


def _agg2_body(ptab, s_uwN, d_uw, s_wu, d_wu, zfeat,
               s2,
               acc, rows, sidx, didx, sem0, sem1, sem2, sem3):
    c = lax.axis_index("c")
    s = lax.axis_index("s")
    r0 = s * _RT
    sems = (sem0, sem1, sem2, sem3)

    _zero_rows(zfeat, acc, r0)
    plsc.subcore_barrier()

    @pl.when(c == 0)
    def _():
        _gather_scatter_dir(s, ptab, s_uwN, d_uw, sidx, didx, rows, acc, sems)

    @pl.when(c == 1)
    def _():
        _gather_scatter_dir(s, ptab, s_wu, d_wu, sidx, didx, rows, acc, sems)

    plsc.subcore_barrier()

    @pl.when(c == 0)
    def _():
        _writeout(acc, s2.at[0], r0)

    @pl.when(c == 1)
    def _():
        _writeout(acc, s2.at[1], r0)


def _make_agg2():
    mesh = plsc.VectorSubcoreMesh(core_axis_name="c", subcore_axis_name="s")
    out_type = jax.ShapeDtypeStruct((2, _NP, 128), jnp.float32)
    scratch = [
        pltpu.VMEM_SHARED((_NP, 128), jnp.float32),
        pltpu.VMEM((2, _B, 128), jnp.float32),
        pltpu.VMEM((_IB, _B), jnp.int32),
        pltpu.VMEM((_IB, _B), jnp.int32),
        pltpu.SemaphoreType.DMA,
        pltpu.SemaphoreType.DMA,
        pltpu.SemaphoreType.DMA,
        pltpu.SemaphoreType.DMA,
    ]
    return pl.kernel(
        _agg2_body,
        out_type=out_type,
        mesh=mesh,
        scratch_types=scratch,
    )


def _comb_body(s0_ref, s1_ref, cnt_ref, h0_ref, h1_ref, l_ref, r_ref, b_ref,
               c2_ref, o_ref, p_ref):
    inv = 1.0 / jnp.maximum(cnt_ref[0][:, :1], 1.0)
    lw = l_ref[0]
    rw = r_ref[0]
    o = (_dot(s0_ref[0] * inv, lw[:128, :])
         + _dot(s1_ref[0] * inv, lw[128:, :])
         + _dot(h0_ref[0], rw[:128, :])
         + _dot(h1_ref[0], rw[128:, :])
         + b_ref[0])
    o = jnp.maximum(o, 0.0)
    o_ref[0] = o
    p = _dot(o, c2_ref[0])
    p_ref[0] = jnp.concatenate(
        [p, jnp.zeros((p.shape[0], 128 - _OUT), jnp.float32)], axis=1)


def _comb(s0, s1, cnt, h0, h1, lw2, rw2, b2, c22):
    return pl.pallas_call(
        _comb_body,
        grid=(2, _N // _BN),
        in_specs=[
            pl.BlockSpec((1, _BN, 128), lambda t, i: (t, i, 0)),
            pl.BlockSpec((1, _BN, 128), lambda t, i: (t, i, 0)),
            pl.BlockSpec((1, _BN, 128), lambda t, i: (t, i, 0)),
            pl.BlockSpec((1, _BN, 128), lambda t, i: (t, i, 0)),
            pl.BlockSpec((1, _BN, 128), lambda t, i: (t, i, 0)),
            pl.BlockSpec((1, _HID, _HID), lambda t, i: (t, 0, 0)),
            pl.BlockSpec((1, _HID, _HID), lambda t, i: (t, 0, 0)),
            pl.BlockSpec((1, 1, _HID), lambda t, i: (t, 0, 0)),
            pl.BlockSpec((1, _HID, _OUT), lambda t, i: (t, 0, 0)),
        ],
        out_specs=[
            pl.BlockSpec((1, _BN, _HID), lambda t, i: (t, i, 0)),
            pl.BlockSpec((1, _BN, 128), lambda t, i: (t, i, 0)),
        ],
        out_shape=[
            jax.ShapeDtypeStruct((2, _N, _HID), jnp.float32),
            jax.ShapeDtypeStruct((2, _N, 128), jnp.float32),
        ],
    )(s0, s1, cnt, h0, h1, lw2, rw2, b2, c22)


def _final_body(t_ref, cnt_ref, o_ref, r_ref, b_ref, z_ref):
    inv = 1.0 / jnp.maximum(cnt_ref[0][:, :1], 1.0)
    s2 = t_ref[0][:, :_OUT]
    z = s2 * inv + _dot(o_ref[0], r_ref[0]) + b_ref[0]
    m = jnp.max(z, axis=1, keepdims=True)
    ez = jnp.exp(z - m)
    lse = jnp.log(jnp.sum(ez, axis=1, keepdims=True))
    z_ref[0] = z - m - lse


def _final(t2, cnt, o, rw2, b2):
    return pl.pallas_call(
        _final_body,
        grid=(2, _N // _BN),
        in_specs=[
            pl.BlockSpec((1, _BN, 128), lambda t, i: (t, i, 0)),
            pl.BlockSpec((1, _BN, 128), lambda t, i: (t, i, 0)),
            pl.BlockSpec((1, _BN, _HID), lambda t, i: (t, i, 0)),
            pl.BlockSpec((1, _HID, _OUT), lambda t, i: (t, 0, 0)),
            pl.BlockSpec((1, 1, _OUT), lambda t, i: (t, 0, 0)),
        ],
        out_specs=pl.BlockSpec((1, _BN, _OUT), lambda t, i: (t, i, 0)),
        out_shape=jax.ShapeDtypeStruct((2, _N, _OUT), jnp.float32),
    )(t2, cnt, o, rw2, b2)


def kernel(x_websites, x_users, ei_u2w, ei_w2u,
           lin_w_web, lin_b_web, lin_w_usr, lin_b_usr,
           c1_uw_l, c1_uw_r, c1_uw_b, c1_wu_l, c1_wu_r, c1_wu_b,
           c2_uw_l, c2_uw_r, c2_uw_b, c2_wu_l, c2_wu_r, c2_wu_b):
    z128 = jnp.zeros((_NP, 128), jnp.float32)
    ones_h = jnp.ones((_B, 128), jnp.float32)
    shp3 = (_NT * _NB, _IB, _B)
    s_uwN = (ei_u2w[0] + _N).reshape(shp3)
    d_uw = ei_u2w[1].reshape(shp3)
    s_wu = ei_w2u[0].reshape(shp3)
    d_wu = ei_w2u[1].reshape(shp3)

    cnt = _make_cnt()(d_uw, d_wu, z128, ones_h)

    x2 = jnp.stack([x_websites, x_users])
    w2 = jnp.stack([lin_w_web, lin_w_usr])
    b2 = jnp.stack([lin_b_web, lin_b_usr]).reshape(2, 1, _HID)
    h0, h1 = _proj(x2, w2, b2)

    s0, s1 = _make_agg1()(
        h0.reshape(2 * _N, 128), h1.reshape(2 * _N, 128),
        s_uwN, d_uw, s_wu, d_wu, z128)

    l2 = jnp.stack([c1_uw_l, c1_wu_l])
    r2 = jnp.stack([c1_uw_r, c1_wu_r])
    cb2 = jnp.stack([c1_uw_b, c1_wu_b]).reshape(2, 1, _HID)
    c22 = jnp.stack([c2_wu_l, c2_uw_l])
    o, p = _comb(s0, s1, cnt, h0, h1, l2, r2, cb2, c22)

    s2 = _make_agg2()(
        p.reshape(2 * _N, 128), s_uwN, d_uw, s_wu, d_wu, z128)

    r22 = jnp.stack([c2_uw_r, c2_wu_r])
    fb2 = jnp.stack([c2_uw_b, c2_wu_b]).reshape(2, 1, _OUT)
    z = _final(s2, cnt, o, r22, fb2)
    return (z[0], z[1])


# speedup vs baseline: 1.0128x; 1.0128x over previous
import jax
import jax.numpy as jnp
from jax import lax
from jax.experimental import pallas as pl
from jax.experimental.pallas import tpu as pltpu
from jax.experimental.pallas import tpu_sc as plsc

_N = 10000
_D_IN = 128
_HID = 256
_OUT = 64
_E = 320000

_NT = 16
_NP = 10240
_RT = _NP // _NT
_B = 125
_ET = _E // _NT
_NCH = _ET // _B
_IB = 40
_NB = _NCH // _IB

_BN = 1000


def _dot(a, b):
    return jnp.dot(a, b, preferred_element_type=jnp.float32)


def _proj_body(x_ref, w_ref, b_ref, h0_ref, h1_ref):
    h = _dot(x_ref[...], w_ref[...]) + b_ref[...]
    h = jnp.maximum(h, 0.0)
    h0_ref[...] = h[:, :128]
    h1_ref[...] = h[:, 128:]


def _proj(x, w, b2):
    return pl.pallas_call(
        _proj_body,
        grid=(_N // _BN,),
        in_specs=[
            pl.BlockSpec((_BN, _D_IN), lambda i: (i, 0)),
            pl.BlockSpec((_D_IN, _HID), lambda i: (0, 0)),
            pl.BlockSpec((1, _HID), lambda i: (0, 0)),
        ],
        out_specs=[
            pl.BlockSpec((_BN, 128), lambda i: (i, 0)),
            pl.BlockSpec((_BN, 128), lambda i: (i, 0)),
        ],
        out_shape=[jax.ShapeDtypeStruct((_N, 128), jnp.float32)] * 2,
    )(x, w, b2)


def _gather_scatter_dir(s, table, src3, dst3, sidx, didx, rows, acc, sems):
    def blk(k, carry):
        pltpu.sync_copy(src3.at[s * _NB + k], sidx)
        pltpu.sync_copy(dst3.at[s * _NB + k], didx)
        pltpu.async_copy(table.at[sidx.at[0]], rows.at[0], sems[0])

        def pair(i, c2):
            for b in (0, 1):
                ch = i * 2 + b
                nb = 1 - b

                @pl.when(ch + 1 < _IB)
                def _():
                    pltpu.async_copy(table.at[sidx.at[ch + 1]], rows.at[nb],
                                     sems[nb])

                pltpu.make_async_copy(table.at[sidx.at[ch]], rows.at[b],
                                      sems[b]).wait()
                pltpu.sync_copy(rows.at[b], acc.at[didx.at[ch]], add=True)
            return c2

        lax.fori_loop(0, _IB // 2, pair, 0)
        return carry

    lax.fori_loop(0, _NB, blk, 0)


_CB = 8


def _count_dir(s, dst3, didx, ones_v, acc, sem):
    def blk(k, carry):
        pltpu.sync_copy(dst3.at[s * _NB + k], didx)

        def burst(i, c2):
            for j in range(_CB):
                pltpu.async_copy(ones_v, acc.at[didx.at[i * _CB + j]], sem,
                                 add=True)
            for j in range(_CB):
                pltpu.make_async_copy(ones_v, acc.at[didx.at[i * _CB + j]],
                                      sem).wait()
            return c2

        lax.fori_loop(0, _IB // _CB, burst, 0)
        return carry

    lax.fori_loop(0, _NB, blk, 0)


def _zero_rows(zfeat, acc, r0):
    pltpu.sync_copy(zfeat.at[pl.ds(r0, _RT), :], acc.at[pl.ds(r0, _RT), :])


def _writeout(acc, out, r0):
    pltpu.sync_copy(acc.at[pl.ds(r0, _RT), :], out.at[pl.ds(r0, _RT), :])


def _agg1_body(hu0, hu1, hw0, hw1, s_uw, d_uw, s_wu, d_wu, zfeat,
               sw0, sw1, su0, su1,
               acc, rows, sidx, didx, sem0, sem1):
    c = lax.axis_index("c")
    s = lax.axis_index("s")
    r0 = s * _RT
    sems = (sem0, sem1)

    _zero_rows(zfeat, acc, r0)
    plsc.subcore_barrier()

    @pl.when(c == 0)
    def _():
        _gather_scatter_dir(s, hu0, s_uw, d_uw, sidx, didx, rows, acc, sems)

    @pl.when(c == 1)
    def _():
        _gather_scatter_dir(s, hu1, s_uw, d_uw, sidx, didx, rows, acc, sems)

    plsc.subcore_barrier()

    @pl.when(c == 0)
    def _():
        _writeout(acc, sw0, r0)

    @pl.when(c == 1)
    def _():
        _writeout(acc, sw1, r0)

    _zero_rows(zfeat, acc, r0)
    plsc.subcore_barrier()

    @pl.when(c == 0)
    def _():
        _gather_scatter_dir(s, hw0, s_wu, d_wu, sidx, didx, rows, acc, sems)

    @pl.when(c == 1)
    def _():
        _gather_scatter_dir(s, hw1, s_wu, d_wu, sidx, didx, rows, acc, sems)

    plsc.subcore_barrier()

    @pl.when(c == 0)
    def _():
        _writeout(acc, su0, r0)

    @pl.when(c == 1)
    def _():
        _writeout(acc, su1, r0)


def _make_agg1():
    mesh = plsc.VectorSubcoreMesh(core_axis_name="c", subcore_axis_name="s")
    out_type = [jax.ShapeDtypeStruct((_NP, 128), jnp.float32)] * 4
    scratch = [
        pltpu.VMEM_SHARED((_NP, 128), jnp.float32),
        pltpu.VMEM((2, _B, 128), jnp.float32),
        pltpu.VMEM((_IB, _B), jnp.int32),
        pltpu.VMEM((_IB, _B), jnp.int32),
        pltpu.SemaphoreType.DMA,
        pltpu.SemaphoreType.DMA,
    ]
    return pl.kernel(
        _agg1_body,
        out_type=out_type,
        mesh=mesh,
        scratch_types=scratch,
    )


def _cnt_body(d_uw, d_wu, zfeat, ones_h,
              cw, cu,
              acc, didx, ones_v, sem0):
    c = lax.axis_index("c")
    s = lax.axis_index("s")
    r0 = s * _RT

    pltpu.sync_copy(ones_h, ones_v)
    _zero_rows(zfeat, acc, r0)
    plsc.subcore_barrier()

    @pl.when(c == 0)
    def _():
        _count_dir(s, d_uw, didx, ones_v, acc, sem0)

    @pl.when(c == 1)
    def _():
        _count_dir(s, d_wu, didx, ones_v, acc, sem0)

    plsc.subcore_barrier()

    @pl.when(c == 0)
    def _():
        _writeout(acc, cw, r0)

    @pl.when(c == 1)
    def _():
        _writeout(acc, cu, r0)


def _make_cnt():
    mesh = plsc.VectorSubcoreMesh(core_axis_name="c", subcore_axis_name="s")
    out_type = [jax.ShapeDtypeStruct((_NP, 128), jnp.float32)] * 2
    scratch = [
        pltpu.VMEM_SHARED((_NP, 128), jnp.float32),
        pltpu.VMEM((_IB, _B), jnp.int32),
        pltpu.VMEM((_B, 128), jnp.float32),
        pltpu.SemaphoreType.DMA,
    ]
    return pl.kernel(
        _cnt_body,
        out_type=out_type,
        mesh=mesh,
        scratch_types=scratch,
    )


def _agg2_body(pu, pw, s_uw, d_uw, s_wu, d_wu, zfeat,
               s2w, s2u,
               acc, rows, sidx, didx, sem0, sem1):
    c = lax.axis_index("c")
    s = lax.axis_index("s")
    r0 = s * _RT
    sems = (sem0, sem1)

    _zero_rows(zfeat, acc, r0)
    plsc.subcore_barrier()

    @pl.when(c == 0)
    def _():
        _gather_scatter_dir(s, pu, s_uw, d_uw, sidx, didx, rows, acc, sems)

    @pl.when(c == 1)
    def _():
        _gather_scatter_dir(s, pw, s_wu, d_wu, sidx, didx, rows, acc, sems)

    plsc.subcore_barrier()

    @pl.when(c == 0)
    def _():
        _writeout(acc, s2w, r0)

    @pl.when(c == 1)
    def _():
        _writeout(acc, s2u, r0)


def _make_agg2():
    mesh = plsc.VectorSubcoreMesh(core_axis_name="c", subcore_axis_name="s")
    out_type = [jax.ShapeDtypeStruct((_NP, 128), jnp.float32)] * 2
    scratch = [
        pltpu.VMEM_SHARED((_NP, 128), jnp.float32),
        pltpu.VMEM((2, _B, 128), jnp.float32),
        pltpu.VMEM((_IB, _B), jnp.int32),
        pltpu.VMEM((_IB, _B), jnp.int32),
        pltpu.SemaphoreType.DMA,
        pltpu.SemaphoreType.DMA,
    ]
    return pl.kernel(
        _agg2_body,
        out_type=out_type,
        mesh=mesh,
        scratch_types=scratch,
    )


def _comb_body(s0_ref, s1_ref, cnt_ref, h0_ref, h1_ref, l_ref, r_ref, b_ref,
               c2_ref, o_ref, p_ref):
    inv = 1.0 / jnp.maximum(cnt_ref[...][:, :1], 1.0)
    lw = l_ref[...]
    rw = r_ref[...]
    o = (_dot(s0_ref[...] * inv, lw[:128, :])
         + _dot(s1_ref[...] * inv, lw[128:, :])
         + _dot(h0_ref[...], rw[:128, :])
         + _dot(h1_ref[...], rw[128:, :])
         + b_ref[...])
    o = jnp.maximum(o, 0.0)
    o_ref[...] = o
    p = _dot(o, c2_ref[...])
    p_ref[...] = jnp.concatenate(
        [p, jnp.zeros((p.shape[0], 128 - _OUT), jnp.float32)], axis=1)


def _comb(s0, s1, cnt, h0, h1, lw, rw, b2, c2):
    return pl.pallas_call(
        _comb_body,
        grid=(_N // _BN,),
        in_specs=[
            pl.BlockSpec((_BN, 128), lambda i: (i, 0)),
            pl.BlockSpec((_BN, 128), lambda i: (i, 0)),
            pl.BlockSpec((_BN, 128), lambda i: (i, 0)),
            pl.BlockSpec((_BN, 128), lambda i: (i, 0)),
            pl.BlockSpec((_BN, 128), lambda i: (i, 0)),
            pl.BlockSpec((_HID, _HID), lambda i: (0, 0)),
            pl.BlockSpec((_HID, _HID), lambda i: (0, 0)),
            pl.BlockSpec((1, _HID), lambda i: (0, 0)),
            pl.BlockSpec((_HID, _OUT), lambda i: (0, 0)),
        ],
        out_specs=[
            pl.BlockSpec((_BN, _HID), lambda i: (i, 0)),
            pl.BlockSpec((_BN, 128), lambda i: (i, 0)),
        ],
        out_shape=[
            jax.ShapeDtypeStruct((_N, _HID), jnp.float32),
            jax.ShapeDtypeStruct((_N, 128), jnp.float32),
        ],
    )(s0, s1, cnt, h0, h1, lw, rw, b2, c2)


def _final_body(t_ref, cnt_ref, o_ref, r_ref, b_ref, z_ref):
    inv = 1.0 / jnp.maximum(cnt_ref[...][:, :1], 1.0)
    s2 = t_ref[...][:, :_OUT]
    z = s2 * inv + _dot(o_ref[...], r_ref[...]) + b_ref[...]
    m = jnp.max(z, axis=1, keepdims=True)
    ez = jnp.exp(z - m)
    lse = jnp.log(jnp.sum(ez, axis=1, keepdims=True))
    z_ref[...] = z - m - lse


def _final(t, cnt, o, rw, b2):
    return pl.pallas_call(
        _final_body,
        grid=(_N // _BN,),
        in_specs=[
            pl.BlockSpec((_BN, 128), lambda i: (i, 0)),
            pl.BlockSpec((_BN, 128), lambda i: (i, 0)),
            pl.BlockSpec((_BN, _HID), lambda i: (i, 0)),
            pl.BlockSpec((_HID, _OUT), lambda i: (0, 0)),
            pl.BlockSpec((1, _OUT), lambda i: (0, 0)),
        ],
        out_specs=pl.BlockSpec((_BN, _OUT), lambda i: (i, 0)),
        out_shape=jax.ShapeDtypeStruct((_N, _OUT), jnp.float32),
    )(t, cnt, o, rw, b2)


def kernel(x_websites, x_users, ei_u2w, ei_w2u,
           lin_w_web, lin_b_web, lin_w_usr, lin_b_usr,
           c1_uw_l, c1_uw_r, c1_uw_b, c1_wu_l, c1_wu_r, c1_wu_b,
           c2_uw_l, c2_uw_r, c2_uw_b, c2_wu_l, c2_wu_r, c2_wu_b):
    z128 = jnp.zeros((_NP, 128), jnp.float32)
    ones_h = jnp.ones((_B, 128), jnp.float32)
    shp3 = (_NT * _NB, _IB, _B)
    s_uw = ei_u2w[0].reshape(shp3)
    d_uw = ei_u2w[1].reshape(shp3)
    s_wu = ei_w2u[0].reshape(shp3)
    d_wu = ei_w2u[1].reshape(shp3)

    cw, cu = _make_cnt()(d_uw, d_wu, z128, ones_h)

    h_w0, h_w1 = _proj(x_websites, lin_w_web, lin_b_web.reshape(1, -1))
    h_u0, h_u1 = _proj(x_users, lin_w_usr, lin_b_usr.reshape(1, -1))

    sw0, sw1, su0, su1 = _make_agg1()(
        h_u0, h_u1, h_w0, h_w1, s_uw, d_uw, s_wu, d_wu, z128)

    o_w, p_w = _comb(sw0, sw1, cw, h_w0, h_w1,
                     c1_uw_l, c1_uw_r, c1_uw_b.reshape(1, -1), c2_wu_l)
    o_u, p_u = _comb(su0, su1, cu, h_u0, h_u1,
                     c1_wu_l, c1_wu_r, c1_wu_b.reshape(1, -1), c2_uw_l)

    s2w, s2u = _make_agg2()(
        p_u, p_w, s_uw, d_uw, s_wu, d_wu, z128)

    z_w = _final(s2w, cw, o_w, c2_uw_r, c2_uw_b.reshape(1, -1))
    z_u = _final(s2u, cu, o_u, c2_wu_r, c2_wu_b.reshape(1, -1))
    return (z_w, z_u)
